# baseline (speedup 1.0000x reference)
import functools

import jax
import jax.numpy as jnp
from jax import lax
from jax.experimental import pallas as pl
from jax.experimental.pallas import tpu as pltpu
from jax.experimental.pallas import tpu_sc as plsc

_ROWS = 8192
_DIM = 1024
_NC = 2
_NS = 16
_NW = _NC * _NS
_ROWS_PER_W = _ROWS // _NW
_CHUNK = 32
_DEPTH = 2
_NCHUNK = _ROWS_PER_W // _CHUNK


def _make_sc_copy():
    mesh = plsc.VectorSubcoreMesh(core_axis_name="c", subcore_axis_name="s")

    @functools.partial(
        pl.kernel,
        mesh=mesh,
        out_type=jax.ShapeDtypeStruct((_ROWS, _DIM), jnp.float32),
        scratch_types=[
            pltpu.MemorySpace.VMEM_SHARED((_NS, _DEPTH, _CHUNK, _DIM), jnp.float32),
            pltpu.SemaphoreType.DMA,
            pltpu.SemaphoreType.DMA,
        ],
    )
    def k(table_hbm, out_hbm, buf, in_sem, out_sem):
        sid = lax.axis_index("s")
        wid = sid * _NC + lax.axis_index("c")
        base = wid * _ROWS_PER_W

        def in_copy(c, slot):
            return pltpu.make_async_copy(
                table_hbm.at[pl.ds(base + c * _CHUNK, _CHUNK)],
                buf.at[sid, slot], in_sem)

        def out_copy(c, slot):
            return pltpu.make_async_copy(
                buf.at[sid, slot],
                out_hbm.at[pl.ds(base + c * _CHUNK, _CHUNK)], out_sem)

        in_copy(0, 0).start()

        def body(c, _):
            slot = lax.rem(c, _DEPTH)
            in_copy(c, slot).wait()
            out_copy(c, slot).start()

            @pl.when(c + 1 < _NCHUNK)
            def _():
                nslot = lax.rem(c + 1, _DEPTH)

                @pl.when(c >= 1)
                def _():
                    out_copy(c - 1, nslot).wait()

                in_copy(c + 1, nslot).start()

            return ()

        lax.fori_loop(0, _NCHUNK, body, (), unroll=False)
        out_copy(_NCHUNK - 2, lax.rem(_NCHUNK - 2, _DEPTH)).wait()
        out_copy(_NCHUNK - 1, lax.rem(_NCHUNK - 1, _DEPTH)).wait()

    return k


_sc_copy = _make_sc_copy()


def kernel(x, emb_weight):
    del x
    return _sc_copy(emb_weight)


# speedup vs baseline: 1.0790x; 1.0790x over previous
import functools

import jax
import jax.numpy as jnp
from jax import lax
from jax.experimental import pallas as pl
from jax.experimental.pallas import tpu as pltpu
from jax.experimental.pallas import tpu_sc as plsc

_ROWS = 8192
_DIM = 1024
_NC = 2
_ROWS_PER_C = _ROWS // _NC
_CHUNK = 512
_DEPTH = 3
_NCHUNK = _ROWS_PER_C // _CHUNK


def _make_sc_copy():
    mesh = plsc.ScalarSubcoreMesh(axis_name="c", num_cores=_NC)

    @functools.partial(
        pl.kernel,
        mesh=mesh,
        out_type=jax.ShapeDtypeStruct((_ROWS, _DIM), jnp.float32),
        scratch_types=[
            pltpu.MemorySpace.VMEM_SHARED((_DEPTH, _CHUNK, _DIM), jnp.float32),
            pltpu.SemaphoreType.DMA,
            pltpu.SemaphoreType.DMA,
        ],
    )
    def k(table_hbm, out_hbm, buf, in_sem, out_sem):
        cid = lax.axis_index("c")
        base = cid * _ROWS_PER_C

        def in_copy(c, slot):
            return pltpu.make_async_copy(
                table_hbm.at[pl.ds(base + c * _CHUNK, _CHUNK)],
                buf.at[slot], in_sem)

        def out_copy(c, slot):
            return pltpu.make_async_copy(
                buf.at[slot],
                out_hbm.at[pl.ds(base + c * _CHUNK, _CHUNK)], out_sem)

        in_copy(0, 0).start()
        in_copy(1, 1).start()

        def body(c, _):
            slot = lax.rem(c, _DEPTH)
            in_copy(c, slot).wait()
            out_copy(c, slot).start()

            @pl.when(c + 2 < _NCHUNK)
            def _():
                nslot = lax.rem(c + 2, _DEPTH)

                @pl.when(c >= 1)
                def _():
                    out_copy(c - 1, nslot).wait()

                in_copy(c + 2, nslot).start()

            return ()

        lax.fori_loop(0, _NCHUNK, body, (), unroll=False)
        out_copy(_NCHUNK - 2, lax.rem(_NCHUNK - 2, _DEPTH)).wait()
        out_copy(_NCHUNK - 1, lax.rem(_NCHUNK - 1, _DEPTH)).wait()

    return k


_sc_copy = _make_sc_copy()


def kernel(x, emb_weight):
    del x
    return _sc_copy(emb_weight)
